# baseline (speedup 1.0000x reference)
import functools

import jax
import jax.numpy as jnp
from jax import lax
from jax.experimental import pallas as pl
from jax.experimental.pallas import tpu as pltpu
from jax.experimental.pallas import tpu_sc as plsc

DIM = 64
L = 200
LANES = 16
NCOL = DIM // LANES
CH0 = 128
CH1 = L - CH0
RED_UNROLL = 8
NUM_CORES = 2
NUM_SUBCORES = 16
NW = NUM_CORES * NUM_SUBCORES


def _make_kernel(B):
    b_per_w = B // NW
    mesh = plsc.VectorSubcoreMesh(core_axis_name="c", subcore_axis_name="s")

    @functools.partial(
        pl.kernel,
        mesh=mesh,
        out_type=jax.ShapeDtypeStruct((B, DIM), jnp.float32),
        compiler_params=pltpu.CompilerParams(use_tc_tiling_on_sc=False),
        scratch_types=[
            pltpu.VMEM((b_per_w * L,), jnp.int32),
            pltpu.VMEM((L, DIM), jnp.float32),
            pltpu.VMEM((L, DIM), jnp.float32),
            pltpu.VMEM((b_per_w, DIM), jnp.float32),
            pltpu.SemaphoreType.DMA,
            pltpu.SemaphoreType.DMA,
        ],
    )
    def k(table_hbm, idx_hbm, out_hbm, idx_v, buf0, buf1, out_v, sem0, sem1):
        wid = lax.axis_index("s") * NUM_CORES + lax.axis_index("c")
        base = wid * b_per_w * L
        pltpu.sync_copy(idx_hbm.at[pl.ds(base, b_per_w * L)], idx_v)

        def start(e, buf, sem):
            off = e * L
            pltpu.async_copy(
                table_hbm.at[idx_v.at[pl.ds(off, CH0)]],
                buf.at[pl.ds(0, CH0), :],
                sem,
            )
            pltpu.async_copy(
                table_hbm.at[idx_v.at[pl.ds(off + CH0, CH1)]],
                buf.at[pl.ds(CH0, CH1), :],
                sem,
            )

        def wait(buf, sem):
            pltpu.make_async_copy(table_hbm.at[pl.ds(0, L), :], buf, sem).wait()

        def reduce_store(e, buf):
            def body(i, carry):
                accs = list(carry)
                r = i * RED_UNROLL
                for u in range(RED_UNROLL):
                    for c in range(NCOL):
                        accs[c] = accs[c] + buf[r + u, pl.ds(c * LANES, LANES)]
                return tuple(accs)

            z = jnp.zeros((LANES,), jnp.float32)
            accs = lax.fori_loop(0, L // RED_UNROLL, body, (z,) * NCOL)
            for c in range(NCOL):
                m = accs[c] * (1.0 / L)
                out_v[e, pl.ds(c * LANES, LANES)] = 1.0 / (1.0 + jnp.exp(-m))

        start(0, buf0, sem0)
        start(1, buf1, sem1)

        def outer(g, carry):
            e0 = 2 * g
            wait(buf0, sem0)
            reduce_store(e0, buf0)

            @pl.when(e0 + 2 < b_per_w)
            def _():
                start(e0 + 2, buf0, sem0)

            wait(buf1, sem1)
            reduce_store(e0 + 1, buf1)

            @pl.when(e0 + 3 < b_per_w)
            def _():
                start(e0 + 3, buf1, sem1)

            return carry

        lax.fori_loop(0, b_per_w // 2, outer, 0)
        pltpu.sync_copy(out_v, out_hbm.at[pl.ds(wid * b_per_w, b_per_w), :])

    return k


def kernel(indices, table):
    B, seq = indices.shape
    assert seq == L and B % NW == 0
    flat = indices.reshape(-1)
    return _make_kernel(B)(table, flat)
